# Initial kernel scaffold; baseline (speedup 1.0000x reference)
import jax
import jax.numpy as jnp
from jax.experimental import pallas as pl


def kernel(x, expert_indices, gate_proj, up_proj, down_proj):
    raise NotImplementedError("write your pallas kernel here")


# speedup vs baseline: 6.9262x; 6.9262x over previous
import jax
import jax.numpy as jnp
from jax.experimental import pallas as pl
from jax.experimental.pallas import tpu as pltpu

_NUM_EXPERTS = 8
_H = 1024
_I = 1024
_T = 64
_IC = 256


def _ffn_body(idx_ref, x_ref, g_ref, u_ref, d_ref, o_ref):
    e = pl.program_id(0)
    c = pl.program_id(1)

    @pl.when(jnp.logical_and(e == 0, c == 0))
    def _init():
        o_ref[...] = jnp.zeros_like(o_ref)

    mask = idx_ref[...] == e
    xm = jnp.where(mask, x_ref[...], 0.0)
    g = jnp.dot(xm, g_ref[0], preferred_element_type=jnp.float32)
    u = jnp.dot(xm, u_ref[0], preferred_element_type=jnp.float32)
    inter = g * jax.nn.sigmoid(g) * u
    o_ref[...] += jax.lax.dot_general(
        inter, d_ref[0], (((1,), (1,)), ((), ())),
        preferred_element_type=jnp.float32)


def kernel(x, expert_indices, gate_proj, up_proj, down_proj):
    b, s, h = x.shape
    x_flat = x.reshape(-1, h)
    idx = expert_indices.reshape(-1, 1).astype(jnp.int32)
    n_c = _I // _IC

    out = pl.pallas_call(
        _ffn_body,
        grid=(_NUM_EXPERTS, n_c),
        in_specs=[
            pl.BlockSpec((_T, 1), lambda e, c: (0, 0)),
            pl.BlockSpec((_T, _H), lambda e, c: (0, 0)),
            pl.BlockSpec((1, _H, _IC), lambda e, c: (e, 0, c)),
            pl.BlockSpec((1, _H, _IC), lambda e, c: (e, 0, c)),
            pl.BlockSpec((1, _H, _IC), lambda e, c: (e, 0, c)),
        ],
        out_specs=pl.BlockSpec((_T, _H), lambda e, c: (0, 0)),
        out_shape=jax.ShapeDtypeStruct((_T, _H), jnp.float32),
        compiler_params=pltpu.CompilerParams(
            dimension_semantics=("arbitrary", "arbitrary"),
        ),
    )(idx, x_flat, gate_proj, up_proj, down_proj)
    return out.reshape(b, s, h)


# speedup vs baseline: 8.5883x; 1.2400x over previous
import jax
import jax.numpy as jnp
from jax.experimental import pallas as pl
from jax.experimental.pallas import tpu as pltpu

_NUM_EXPERTS = 8
_H = 1024
_I = 1024
_T = 64
_IC = 512


def _ffn_body(idx_ref, x_ref, g_ref, u_ref, d_ref, o_ref):
    e = pl.program_id(0)
    c = pl.program_id(1)

    @pl.when(jnp.logical_and(e == 0, c == 0))
    def _init():
        o_ref[...] = jnp.zeros_like(o_ref)

    mask = idx_ref[...] == e
    xm = jnp.where(mask, x_ref[...], 0.0)
    g = jnp.dot(xm, g_ref[0], preferred_element_type=jnp.float32)
    u = jnp.dot(xm, u_ref[0], preferred_element_type=jnp.float32)
    inter = g * jax.nn.sigmoid(g) * u
    o_ref[...] += jax.lax.dot_general(
        inter, d_ref[0], (((1,), (1,)), ((), ())),
        preferred_element_type=jnp.float32)


def kernel(x, expert_indices, gate_proj, up_proj, down_proj):
    b, s, h = x.shape
    x_flat = x.reshape(-1, h)
    idx = expert_indices.reshape(-1, 1).astype(jnp.int32)
    n_c = _I // _IC

    out = pl.pallas_call(
        _ffn_body,
        grid=(_NUM_EXPERTS, n_c),
        in_specs=[
            pl.BlockSpec((_T, 1), lambda e, c: (0, 0)),
            pl.BlockSpec((_T, _H), lambda e, c: (0, 0)),
            pl.BlockSpec((1, _H, _IC), lambda e, c: (e, 0, c)),
            pl.BlockSpec((1, _H, _IC), lambda e, c: (e, 0, c)),
            pl.BlockSpec((1, _H, _IC), lambda e, c: (e, 0, c)),
        ],
        out_specs=pl.BlockSpec((_T, _H), lambda e, c: (0, 0)),
        out_shape=jax.ShapeDtypeStruct((_T, _H), jnp.float32),
        compiler_params=pltpu.CompilerParams(
            dimension_semantics=("arbitrary", "arbitrary"),
        ),
    )(idx, x_flat, gate_proj, up_proj, down_proj)
    return out.reshape(b, s, h)


# speedup vs baseline: 8.9164x; 1.0382x over previous
import jax
import jax.numpy as jnp
from jax.experimental import pallas as pl
from jax.experimental.pallas import tpu as pltpu

_NUM_EXPERTS = 8
_H = 1024
_I = 1024
_T = 64
_IC = 1024


def _ffn_body(idx_ref, x_ref, g_ref, u_ref, d_ref, o_ref):
    e = pl.program_id(0)
    c = pl.program_id(1)

    @pl.when(jnp.logical_and(e == 0, c == 0))
    def _init():
        o_ref[...] = jnp.zeros_like(o_ref)

    mask = idx_ref[...] == e
    xm = jnp.where(mask, x_ref[...], 0.0)
    g = jnp.dot(xm, g_ref[0], preferred_element_type=jnp.float32)
    u = jnp.dot(xm, u_ref[0], preferred_element_type=jnp.float32)
    inter = g * jax.nn.sigmoid(g) * u
    o_ref[...] += jax.lax.dot_general(
        inter, d_ref[0], (((1,), (1,)), ((), ())),
        preferred_element_type=jnp.float32)


def kernel(x, expert_indices, gate_proj, up_proj, down_proj):
    b, s, h = x.shape
    x_flat = x.reshape(-1, h)
    idx = expert_indices.reshape(-1, 1).astype(jnp.int32)
    n_c = _I // _IC

    out = pl.pallas_call(
        _ffn_body,
        grid=(_NUM_EXPERTS, n_c),
        in_specs=[
            pl.BlockSpec((_T, 1), lambda e, c: (0, 0)),
            pl.BlockSpec((_T, _H), lambda e, c: (0, 0)),
            pl.BlockSpec((1, _H, _IC), lambda e, c: (e, 0, c)),
            pl.BlockSpec((1, _H, _IC), lambda e, c: (e, 0, c)),
            pl.BlockSpec((1, _H, _IC), lambda e, c: (e, 0, c)),
        ],
        out_specs=pl.BlockSpec((_T, _H), lambda e, c: (0, 0)),
        out_shape=jax.ShapeDtypeStruct((_T, _H), jnp.float32),
        compiler_params=pltpu.CompilerParams(
            dimension_semantics=("arbitrary", "arbitrary"),
        ),
    )(idx, x_flat, gate_proj, up_proj, down_proj)
    return out.reshape(b, s, h)
